# baseline (speedup 1.0000x reference)
import functools

import jax
import jax.numpy as jnp
from jax import lax
from jax.experimental import pallas as pl
from jax.experimental.pallas import tpu as pltpu
from jax.experimental.pallas import tpu_sc as plsc

EMB = 4096
NE = 64
K = 8
NT = 8192
M_BLK = 1024

NW = 32
TPW = NT // NW
GRP = TPW // 16
GI = 4


def _mm_block(x_ref, w_ref, scores_ref):
    scores_ref[...] = jax.lax.dot_general(
        x_ref[...], w_ref[...], (((1,), (1,)), ((), ())),
        preferred_element_type=jnp.float32,
    )


def _gate_scores(x, w):
    grid = (NT // M_BLK,)
    return pl.pallas_call(
        _mm_block,
        grid=grid,
        in_specs=[
            pl.BlockSpec((M_BLK, EMB), lambda i: (i, 0)),
            pl.BlockSpec((NE, EMB), lambda i: (0, 0)),
        ],
        out_specs=pl.BlockSpec((M_BLK, NE), lambda i: (i, 0)),
        out_shape=jax.ShapeDtypeStruct((NT, NE), jnp.float32),
    )(x, w)


def _topk_body(scores_hbm, probs_hbm, idx_hbm, svmem, pvmem, ivmem):
    wid = lax.axis_index("s") * 2 + lax.axis_index("c")
    pltpu.sync_copy(scores_hbm.at[pl.ds(wid * TPW * NE, TPW * NE)], svmem)

    lane = lax.iota(jnp.int32, 16)
    lane64 = lane * NE
    lane8 = lane * K
    m6 = jnp.int32(NE - 1)
    neg_inf = jnp.full((16,), -jnp.inf, jnp.float32)

    def set_body(si, _):
        rb64 = [(si * GI + k) * 16 * NE + lane64 for k in range(GI)]
        rb8 = [(si * GI + k) * 16 * K + lane8 for k in range(GI)]

        def exp_body(e, ts):
            ts = list(ts)
            tie_base = jnp.int32(NE - 1) - e
            for k in range(GI):
                v = plsc.load_gather(svmem, [rb64[k] + e])
                i = plsc.bitcast(v, jnp.int32)
                sgn = lax.shift_right_arithmetic(i, 31)
                key = plsc.bitcast((i & ~m6) | (tie_base ^ (sgn & m6)), jnp.float32)
                for j in range(K):
                    t = ts[k * K + j]
                    ts[k * K + j] = jnp.maximum(t, key)
                    key = jnp.minimum(t, key)
            return tuple(ts)

        ts = lax.fori_loop(0, NE, exp_body, (neg_inf,) * (GI * K), unroll=4)

        for k in range(GI):
            tb = [plsc.bitcast(ts[k * K + j], jnp.int32) for j in range(K)]
            vals = [plsc.bitcast(b & ~m6, jnp.float32) for b in tb]
            es = [jnp.exp(v - vals[0]) for v in vals]
            tot = es[0]
            for j in range(1, K):
                tot = tot + es[j]
            for j in range(K):
                idx_j = (tb[j] & m6) ^ (~lax.shift_right_arithmetic(tb[j], 31) & m6)
                plsc.store_scatter(pvmem, [rb8[k] + j], es[j] / tot)
                plsc.store_scatter(ivmem, [rb8[k] + j], idx_j)
        return 0

    lax.fori_loop(0, GRP // GI, set_body, 0)
    pltpu.sync_copy(pvmem, probs_hbm.at[pl.ds(wid * TPW * K, TPW * K)])
    pltpu.sync_copy(ivmem, idx_hbm.at[pl.ds(wid * TPW * K, TPW * K)])


_topk_sc = functools.partial(
    pl.kernel,
    out_type=[
        jax.ShapeDtypeStruct((NT * K,), jnp.float32),
        jax.ShapeDtypeStruct((NT * K,), jnp.int32),
    ],
    mesh=plsc.VectorSubcoreMesh(core_axis_name="c", subcore_axis_name="s"),
    compiler_params=pltpu.CompilerParams(needs_layout_passes=False),
    scratch_types=[
        pltpu.VMEM((TPW * NE,), jnp.float32),
        pltpu.VMEM((TPW * K,), jnp.float32),
        pltpu.VMEM((TPW * K,), jnp.int32),
    ],
)(_topk_body)


@jax.jit
def kernel(x, W_gate):
    scores = _gate_scores(x, W_gate)
    probs_flat, idx_flat = _topk_sc(scores.reshape(NT * NE))
    return (probs_flat.reshape(NT, K), idx_flat.reshape(NT, K), scores)
